# baseline (speedup 1.0000x reference)
import functools

import jax
import jax.numpy as jnp
from jax import lax
from jax.experimental import pallas as pl
from jax.experimental.pallas import tpu as pltpu
from jax.experimental.pallas import tpu_sc as plsc

NUM_EXPERTS = 8
TOP_K = 2
BLK = 2048
N_TOKENS = 32768
NUM_WORKERS = 32
CHUNK = N_TOKENS // NUM_WORKERS
LANES = 16


def _tc_body(x_ref, wt_ref, b_ref, logits_ref, probs_ref):
    x = x_ref[...]
    wt = wt_ref[...]
    logits = jnp.dot(x, wt, preferred_element_type=jnp.float32) + b_ref[...]

    lt = logits.T
    logits_ref[...] = lt
    m = jnp.max(lt, axis=0, keepdims=True)
    e = jnp.exp(lt - m)
    s = jnp.sum(e, axis=0, keepdims=True)
    probs_ref[...] = e / s


@functools.partial(
    pl.kernel,
    mesh=plsc.VectorSubcoreMesh(core_axis_name="c", subcore_axis_name="s"),
    out_type=[
        jax.ShapeDtypeStruct((TOP_K, N_TOKENS), jnp.float32),
        jax.ShapeDtypeStruct((TOP_K, N_TOKENS), jnp.int32),
    ],
    scratch_types=[
        pltpu.VMEM((NUM_EXPERTS, CHUNK), jnp.float32),
        pltpu.VMEM((TOP_K, CHUNK), jnp.float32),
        pltpu.VMEM((TOP_K, CHUNK), jnp.int32),
    ],
)
def _sc_router(probs_hbm, w_hbm, idx_hbm, p_v, w_v, i_v):
    wid = lax.axis_index("s") * 2 + lax.axis_index("c")
    base = wid * CHUNK
    pltpu.sync_copy(probs_hbm.at[:, pl.ds(base, CHUNK)], p_v)

    def body(j, carry):
        o = j * LANES
        m1 = p_v[0, pl.ds(o, LANES)]
        i1 = jnp.zeros((LANES,), jnp.int32)
        for e in range(1, NUM_EXPERTS):
            pe = p_v[e, pl.ds(o, LANES)]
            better = pe > m1
            m1 = jnp.where(better, pe, m1)
            i1 = jnp.where(better, jnp.int32(e), i1)
        m2 = jnp.full((LANES,), -1.0, jnp.float32)
        i2 = jnp.zeros((LANES,), jnp.int32)
        for e in range(NUM_EXPERTS):
            pe = p_v[e, pl.ds(o, LANES)]
            pe = jnp.where(i1 == e, -1.0, pe)
            better = pe > m2
            m2 = jnp.where(better, pe, m2)
            i2 = jnp.where(better, jnp.int32(e), i2)
        t = jnp.exp(m2 - m1)
        denom = 1.0 + t
        w_v[0, pl.ds(o, LANES)] = 1.0 / denom
        w_v[1, pl.ds(o, LANES)] = t / denom
        i_v[0, pl.ds(o, LANES)] = i1
        i_v[1, pl.ds(o, LANES)] = i2
        return carry

    lax.fori_loop(0, CHUNK // LANES, body, 0)
    pltpu.sync_copy(w_v, w_hbm.at[:, pl.ds(base, CHUNK)])
    pltpu.sync_copy(i_v, idx_hbm.at[:, pl.ds(base, CHUNK)])


@jax.jit
def kernel(inputs, W, b):
    B, S, D = inputs.shape
    N = B * S
    x2d = inputs.reshape(N, D)
    wt = W.T
    b2d = b.reshape(1, NUM_EXPERTS)

    grid = (N // BLK,)
    logits_t, probs_t = pl.pallas_call(
        _tc_body,
        grid=grid,
        in_specs=[
            pl.BlockSpec((BLK, D), lambda i: (i, 0)),
            pl.BlockSpec((D, NUM_EXPERTS), lambda i: (0, 0)),
            pl.BlockSpec((1, NUM_EXPERTS), lambda i: (0, 0)),
        ],
        out_specs=[
            pl.BlockSpec((NUM_EXPERTS, BLK), lambda i: (0, i)),
            pl.BlockSpec((NUM_EXPERTS, BLK), lambda i: (0, i)),
        ],
        out_shape=[
            jax.ShapeDtypeStruct((NUM_EXPERTS, N), jnp.float32),
            jax.ShapeDtypeStruct((NUM_EXPERTS, N), jnp.float32),
        ],
    )(x2d, wt, b2d)

    w_t, idx_t = _sc_router(probs_t)

    return (
        logits_t.T.reshape(B, S, NUM_EXPERTS),
        probs_t.T.reshape(B, S, NUM_EXPERTS),
        w_t.T.reshape(B, S, TOP_K),
        idx_t.T.reshape(B, S, TOP_K),
    )


# speedup vs baseline: 1.3293x; 1.3293x over previous
import jax
import jax.numpy as jnp
from jax.experimental import pallas as pl
from jax.experimental.pallas import tpu as pltpu

NUM_EXPERTS = 8
TOP_K = 2
BLK = 2048


def _router_body(x_ref, wt_ref, b_ref, logits_ref, probs_ref, wi_ref):
    x = x_ref[...]
    wt = wt_ref[...]
    logits = jnp.dot(x, wt, preferred_element_type=jnp.float32) + b_ref[...]

    lt = logits.T
    logits_ref[...] = lt
    m = jnp.max(lt, axis=0, keepdims=True)
    e = jnp.exp(lt - m)
    s = jnp.sum(e, axis=0, keepdims=True)
    pt = e / s
    probs_ref[...] = pt

    eids = jax.lax.broadcasted_iota(jnp.int32, pt.shape, 0)
    p1 = jnp.max(pt, axis=0, keepdims=True)
    i1 = jnp.min(jnp.where(pt == p1, eids, NUM_EXPERTS), axis=0,
                 keepdims=True)
    pt2 = jnp.where(eids == i1, -1.0, pt)
    p2 = jnp.max(pt2, axis=0, keepdims=True)
    i2 = jnp.min(jnp.where(pt2 == p2, eids, NUM_EXPERTS), axis=0,
                 keepdims=True)

    t = jnp.exp(p2 - p1)
    denom = 1.0 + t
    wi_ref[...] = jnp.concatenate(
        [1.0 / denom, t / denom,
         jax.lax.bitcast_convert_type(i1, jnp.float32),
         jax.lax.bitcast_convert_type(i2, jnp.float32)], axis=0)


@jax.jit
def kernel(inputs, W, b):
    B, S, D = inputs.shape
    N = B * S
    x2d = inputs.reshape(N, D)
    wt = W.T
    b2d = b.reshape(1, NUM_EXPERTS)

    grid = (N // BLK,)
    logits_t, probs_t, wi_t = pl.pallas_call(
        _router_body,
        grid=grid,
        in_specs=[
            pl.BlockSpec((BLK, D), lambda i: (i, 0)),
            pl.BlockSpec((D, NUM_EXPERTS), lambda i: (0, 0)),
            pl.BlockSpec((1, NUM_EXPERTS), lambda i: (0, 0)),
        ],
        out_specs=[
            pl.BlockSpec((NUM_EXPERTS, BLK), lambda i: (0, i)),
            pl.BlockSpec((NUM_EXPERTS, BLK), lambda i: (0, i)),
            pl.BlockSpec((2 * TOP_K, BLK), lambda i: (0, i)),
        ],
        out_shape=[
            jax.ShapeDtypeStruct((NUM_EXPERTS, N), jnp.float32),
            jax.ShapeDtypeStruct((NUM_EXPERTS, N), jnp.float32),
            jax.ShapeDtypeStruct((2 * TOP_K, N), jnp.float32),
        ],
    )(x2d, wt, b2d)

    wi = wi_t.T
    return (
        logits_t.T.reshape(B, S, NUM_EXPERTS),
        probs_t.T.reshape(B, S, NUM_EXPERTS),
        wi[:, :TOP_K].reshape(B, S, TOP_K),
        jax.lax.bitcast_convert_type(
            wi[:, TOP_K:], jnp.int32).reshape(B, S, TOP_K),
    )


# speedup vs baseline: 1.4144x; 1.0640x over previous
import jax
import jax.numpy as jnp
from jax.experimental import pallas as pl
from jax.experimental.pallas import tpu as pltpu

NUM_EXPERTS = 8
TOP_K = 2
BLK = 2048


def _router_body(x_ref, wt_ref, b_ref, logits_ref, probs_ref, w_ref, idx_ref):
    x = x_ref[...]
    wt = wt_ref[...]
    logits = jnp.dot(x, wt, preferred_element_type=jnp.float32) + b_ref[...]

    lt = logits.T
    logits_ref[...] = lt
    m = jnp.max(lt, axis=0, keepdims=True)
    e = jnp.exp(lt - m)
    s = jnp.sum(e, axis=0, keepdims=True)
    pt = e / s
    probs_ref[...] = pt

    eids = jax.lax.broadcasted_iota(jnp.int32, pt.shape, 0)
    p1 = jnp.max(pt, axis=0, keepdims=True)
    i1 = jnp.min(jnp.where(pt == p1, eids, NUM_EXPERTS), axis=0,
                 keepdims=True)
    pt2 = jnp.where(eids == i1, -1.0, pt)
    p2 = jnp.max(pt2, axis=0, keepdims=True)
    i2 = jnp.min(jnp.where(pt2 == p2, eids, NUM_EXPERTS), axis=0,
                 keepdims=True)

    t = jnp.exp(p2 - p1)
    denom = 1.0 + t
    w_ref[...] = jnp.concatenate([1.0 / denom, t / denom], axis=0)
    idx_ref[...] = jnp.concatenate([i1, i2], axis=0)


@jax.jit
def kernel(inputs, W, b):
    B, S, D = inputs.shape
    N = B * S
    x2d = inputs.reshape(N, D)
    wt = W.T
    b2d = b.reshape(1, NUM_EXPERTS)

    grid = (N // BLK,)
    logits_t, probs_t, w_t, idx_t = pl.pallas_call(
        _router_body,
        grid=grid,
        in_specs=[
            pl.BlockSpec((BLK, D), lambda i: (i, 0)),
            pl.BlockSpec((D, NUM_EXPERTS), lambda i: (0, 0)),
            pl.BlockSpec((1, NUM_EXPERTS), lambda i: (0, 0)),
        ],
        out_specs=[
            pl.BlockSpec((NUM_EXPERTS, BLK), lambda i: (0, i)),
            pl.BlockSpec((NUM_EXPERTS, BLK), lambda i: (0, i)),
            pl.BlockSpec((TOP_K, BLK), lambda i: (0, i)),
            pl.BlockSpec((TOP_K, BLK), lambda i: (0, i)),
        ],
        out_shape=[
            jax.ShapeDtypeStruct((NUM_EXPERTS, N), jnp.float32),
            jax.ShapeDtypeStruct((NUM_EXPERTS, N), jnp.float32),
            jax.ShapeDtypeStruct((TOP_K, N), jnp.float32),
            jax.ShapeDtypeStruct((TOP_K, N), jnp.int32),
        ],
    )(x2d, wt, b2d)

    return (
        logits_t.T.reshape(B, S, NUM_EXPERTS),
        probs_t.T.reshape(B, S, NUM_EXPERTS),
        w_t.T.reshape(B, S, TOP_K),
        idx_t.T.reshape(B, S, TOP_K),
    )


# speedup vs baseline: 1.4211x; 1.0048x over previous
import jax
import jax.numpy as jnp
from jax.experimental import pallas as pl

NUM_EXPERTS = 8
TOP_K = 2
BLK = 2048


def _router_body(x_ref, wt_ref, b_ref, logits_ref, probs_ref, w_ref, idx_ref):
    x = x_ref[...]
    wt = wt_ref[...]
    logits = jnp.dot(x, wt, preferred_element_type=jnp.float32) + b_ref[...]

    lt = logits.T
    logits_ref[...] = lt
    m = jnp.max(lt, axis=0, keepdims=True)
    e = jnp.exp(lt - m)
    s = jnp.sum(e, axis=0, keepdims=True)
    pt = e / s
    probs_ref[...] = pt

    eids = jax.lax.broadcasted_iota(jnp.int32, pt.shape, 0)
    p1 = jnp.max(pt, axis=0, keepdims=True)
    i1 = jnp.min(jnp.where(pt == p1, eids, NUM_EXPERTS), axis=0,
                 keepdims=True)
    pt2 = jnp.where(eids == i1, -1.0, pt)
    p2 = jnp.max(pt2, axis=0, keepdims=True)
    i2 = jnp.min(jnp.where(pt2 == p2, eids, NUM_EXPERTS), axis=0,
                 keepdims=True)

    t = jnp.exp(p2 - p1)
    denom = 1.0 + t
    w_ref[...] = jnp.concatenate([1.0 / denom, t / denom], axis=0)
    idx_ref[...] = jnp.concatenate([i1, i2], axis=0)


@jax.jit
def kernel(inputs, W, b):
    B, S, D = inputs.shape
    N = B * S
    x2d = inputs.reshape(N, D)
    wt = W.T
    b2d = b.reshape(1, NUM_EXPERTS)

    grid = (N // BLK,)
    logits_t, probs_t, w_t, idx_t = pl.pallas_call(
        _router_body,
        grid=grid,
        in_specs=[
            pl.BlockSpec((BLK, D), lambda i: (i, 0)),
            pl.BlockSpec((D, NUM_EXPERTS), lambda i: (0, 0)),
            pl.BlockSpec((1, NUM_EXPERTS), lambda i: (0, 0)),
        ],
        out_specs=[
            pl.BlockSpec((NUM_EXPERTS, BLK), lambda i: (0, i)),
            pl.BlockSpec((NUM_EXPERTS, BLK), lambda i: (0, i)),
            pl.BlockSpec((TOP_K, BLK), lambda i: (0, i)),
            pl.BlockSpec((TOP_K, BLK), lambda i: (0, i)),
        ],
        out_shape=[
            jax.ShapeDtypeStruct((NUM_EXPERTS, N), jnp.float32),
            jax.ShapeDtypeStruct((NUM_EXPERTS, N), jnp.float32),
            jax.ShapeDtypeStruct((TOP_K, N), jnp.float32),
            jax.ShapeDtypeStruct((TOP_K, N), jnp.int32),
        ],
    )(x2d, wt, b2d)

    return (
        logits_t.T.reshape(B, S, NUM_EXPERTS),
        probs_t.T.reshape(B, S, NUM_EXPERTS),
        w_t.T.reshape(B, S, TOP_K),
        idx_t.T.reshape(B, S, TOP_K),
    )
